# baseline (speedup 1.0000x reference)
import functools

import jax
import jax.numpy as jnp
from jax import lax
from jax.experimental import pallas as pl
from jax.experimental.pallas import tpu as pltpu
from jax.experimental.pallas import tpu_sc as plsc

N = 10000
E = 320000
D = 128
DE = 16
EW = 24

NC = 2
NS = 16
NW = NC * NS
EP = E // NW
C = 80
NCHUNK = EP // C
NPAD = 10112
RPT = NPAD // NS
L = 16
NSLOT = 4


def _sc_body(h_hbm, ei_hbm, ea_hbm,
             aggh_hbm, agge_hbm,
             aggh_sh, agge_sh,
             ei_v, ea_v, rows_v,
             idx_sem, gat_sem, sca_sem):
    c = lax.axis_index("c")
    s = lax.axis_index("s")
    wid = c * NS + s

    z16 = jnp.zeros((L,), jnp.float32)

    def _zrow(i, _):
        rows_v[0, i // 8, pl.ds((i % 8) * L, L)] = z16
        return 0
    lax.fori_loop(0, C * 8, _zrow, 0)

    def _zea(i, _):
        ea_v[0, i, pl.ds(0, L)] = z16
        ea_v[0, i, pl.ds(8, L)] = z16
        return 0
    lax.fori_loop(0, C, _zea, 0)

    for k in range(RPT // C):
        rws = pl.ds(s * RPT + k * C, C)
        pltpu.sync_copy(rows_v.at[0], aggh_sh.at[rws])
        pltpu.sync_copy(ea_v.at[0], agge_sh.at[rws])
    TAILR = RPT - (RPT // C) * C
    if TAILR:
        rws = pl.ds(s * RPT + (RPT // C) * C, TAILR)
        pltpu.sync_copy(rows_v.at[0, pl.ds(0, TAILR)], aggh_sh.at[rws])
        pltpu.sync_copy(ea_v.at[0, pl.ds(0, TAILR)], agge_sh.at[rws])

    plsc.subcore_barrier()

    ebase = wid * EP

    def _issue_idx(g, sb):
        off = ebase + g * C
        pltpu.async_copy(ei_hbm.at[:, pl.ds(off, C)], ei_v.at[sb],
                         idx_sem.at[sb])
        pltpu.async_copy(ea_hbm.at[pl.ds(off, C)], ea_v.at[sb],
                         idx_sem.at[sb])

    def _wait_idx(sb):
        pltpu.make_async_copy(ei_hbm.at[:, pl.ds(0, C)], ei_v.at[sb],
                              idx_sem.at[sb]).wait()
        pltpu.make_async_copy(ea_hbm.at[pl.ds(0, C)], ea_v.at[sb],
                              idx_sem.at[sb]).wait()

    def _issue_gather(rb, sb):
        pltpu.async_copy(h_hbm.at[ei_v.at[sb, 0]], rows_v.at[rb],
                         gat_sem.at[rb])

    def _wait_gather(rb, sb):
        pltpu.make_async_copy(h_hbm.at[ei_v.at[sb, 0]], rows_v.at[rb],
                              gat_sem.at[rb]).wait()

    def _issue_scatter(rb, sb):
        pltpu.async_copy(rows_v.at[rb], aggh_sh.at[ei_v.at[sb, 1]],
                         sca_sem.at[sb], add=True)
        pltpu.async_copy(ea_v.at[sb], agge_sh.at[ei_v.at[sb, 1]],
                         sca_sem.at[sb], add=True)

    def _drain_scatter(rb, sb):
        pltpu.make_async_copy(rows_v.at[rb], aggh_sh.at[ei_v.at[sb, 1]],
                              sca_sem.at[sb]).wait()
        pltpu.make_async_copy(ea_v.at[sb], agge_sh.at[ei_v.at[sb, 1]],
                              sca_sem.at[sb]).wait()

    _issue_idx(0, 0)
    _issue_idx(1, 1)
    _wait_idx(0)
    _issue_gather(0, 0)

    def _chunk(g, _):
        rb = g % 2
        nrb = (g + 1) % 2
        sb = g % NSLOT

        @pl.when(g > 0)
        def _():
            _drain_scatter(nrb, (g - 1) % NSLOT)

        @pl.when(g + 1 < NCHUNK)
        def _():
            _wait_idx((g + 1) % NSLOT)
            _issue_gather(nrb, (g + 1) % NSLOT)

        _wait_gather(rb, sb)
        _issue_scatter(rb, sb)

        @pl.when(g + 2 < NCHUNK)
        def _():
            _issue_idx(g + 2, (g + 2) % NSLOT)
        return 0
    lax.fori_loop(0, NCHUNK, _chunk, 0)

    _drain_scatter((NCHUNK - 1) % 2, (NCHUNK - 1) % NSLOT)
    plsc.subcore_barrier()

    rows = pl.ds(s * RPT, RPT)
    pltpu.sync_copy(aggh_sh.at[rows], aggh_hbm.at[c, rows])
    pltpu.sync_copy(agge_sh.at[rows], agge_hbm.at[c, rows])


def _sc_scatter(h, eiT, eax):
    mesh = plsc.VectorSubcoreMesh(core_axis_name="c", subcore_axis_name="s")
    fn = functools.partial(
        pl.kernel, mesh=mesh,
        out_type=[
            jax.ShapeDtypeStruct((NC, NPAD, D), jnp.float32),
            jax.ShapeDtypeStruct((NC, NPAD, EW), jnp.float32),
        ],
        scratch_types=[
            pltpu.VMEM_SHARED((NPAD, D), jnp.float32),
            pltpu.VMEM_SHARED((NPAD, EW), jnp.float32),
            pltpu.VMEM((NSLOT, 2, C), jnp.int32),
            pltpu.VMEM((NSLOT, C, EW), jnp.float32),
            pltpu.VMEM((2, C, D), jnp.float32),
            pltpu.SemaphoreType.DMA((NSLOT,)),
            pltpu.SemaphoreType.DMA((2,)),
            pltpu.SemaphoreType.DMA((NSLOT,)),
        ],
        compiler_params=pltpu.CompilerParams(use_tc_tiling_on_sc=False,
                                             needs_layout_passes=False),
    )(_sc_body)
    return fn(h, eiT, eax)


def _mlp_body(h_ref, ah_ref, ae_ref, wea_ref,
              w1_ref, b1_ref, w2_ref, b2_ref, o_ref):
    x = h_ref[...] + ah_ref[0] + ah_ref[1]
    e = ae_ref[0] + ae_ref[1]
    u = x + jnp.dot(e, wea_ref[...], preferred_element_type=jnp.float32)
    v = jnp.dot(u, w1_ref[...], preferred_element_type=jnp.float32) + b1_ref[...]
    v = 0.5 * v * (1.0 + lax.erf(v * 0.7071067811865476))
    o_ref[...] = jnp.dot(v, w2_ref[...], preferred_element_type=jnp.float32) + b2_ref[...]


def _tc_mlp(h, aggh, agge, We_aug, W1, b1, W2, b2):
    R = 1000
    grid = (N // R,)
    return pl.pallas_call(
        _mlp_body,
        grid=grid,
        in_specs=[
            pl.BlockSpec((R, D), lambda i: (i, 0)),
            pl.BlockSpec((NC, R, D), lambda i: (0, i, 0)),
            pl.BlockSpec((NC, R, EW), lambda i: (0, i, 0)),
            pl.BlockSpec((EW, D), lambda i: (0, 0)),
            pl.BlockSpec((D, D), lambda i: (0, 0)),
            pl.BlockSpec((1, D), lambda i: (0, 0)),
            pl.BlockSpec((D, D), lambda i: (0, 0)),
            pl.BlockSpec((1, D), lambda i: (0, 0)),
        ],
        out_specs=pl.BlockSpec((R, D), lambda i: (i, 0)),
        out_shape=jax.ShapeDtypeStruct((N, D), jnp.float32),
    )(h, aggh, agge, We_aug, W1, b1, W2, b2)


def kernel(h, edge_index, edge_attr, We, be, W1, b1, W2, b2):
    eiT = edge_index.T
    eax = jnp.concatenate(
        [edge_attr, jnp.ones((E, 1), jnp.float32),
         jnp.zeros((E, EW - DE - 1), jnp.float32)], axis=1)
    aggh, agge = _sc_scatter(h, eiT, eax)
    We_aug = jnp.concatenate(
        [We, be[None, :], jnp.zeros((EW - DE - 1, D), jnp.float32)], axis=0)
    return _tc_mlp(h, aggh, agge, We_aug, W1, b1.reshape(1, D),
                   W2, b2.reshape(1, D))


# speedup vs baseline: 1.7154x; 1.7154x over previous
import functools

import jax
import jax.numpy as jnp
from jax import lax
from jax.experimental import pallas as pl
from jax.experimental.pallas import tpu as pltpu
from jax.experimental.pallas import tpu_sc as plsc

N = 10000
E = 320000
D = 128
DE = 16
EW = 24

NC = 2
NS = 16
NW = NC * NS
EP = E // NW
C = 80
NCHUNK = EP // C
NPAD = 10112
RPT = NPAD // NS
L = 16
NSLOT = 4


def _sc_body(h_hbm, ei_hbm, ea_hbm,
             aggh_hbm, agge_hbm,
             aggh_sh, agge_sh,
             ei_v, eat_v, ea_v, rows_v,
             idx_sem, gat_sem, sca_sem):
    c = lax.axis_index("c")
    s = lax.axis_index("s")
    wid = c * NS + s

    z16 = jnp.zeros((L,), jnp.float32)

    def _zrow(i, _):
        rows_v[0, i // 8, pl.ds((i % 8) * L, L)] = z16
        return 0
    lax.fori_loop(0, C * 8, _zrow, 0)

    def _zea(i, _):
        b = i // C
        r = i % C
        ea_v[b, r, pl.ds(0, L)] = z16
        ea_v[b, r, pl.ds(8, L)] = z16
        return 0
    lax.fori_loop(0, NSLOT * C, _zea, 0)

    for k in range(RPT // C):
        rws = pl.ds(s * RPT + k * C, C)
        pltpu.sync_copy(rows_v.at[0], aggh_sh.at[rws])
        pltpu.sync_copy(ea_v.at[0], agge_sh.at[rws])
    TAILR = RPT - (RPT // C) * C
    if TAILR:
        rws = pl.ds(s * RPT + (RPT // C) * C, TAILR)
        pltpu.sync_copy(rows_v.at[0, pl.ds(0, TAILR)], aggh_sh.at[rws])
        pltpu.sync_copy(ea_v.at[0, pl.ds(0, TAILR)], agge_sh.at[rws])

    lane = lax.iota(jnp.int32, L)
    o16 = jnp.ones((L,), jnp.float32)
    col_deg = jnp.full((L,), DE, jnp.int32)
    for b in range(NSLOT):
        for e0 in range(0, C, L):
            plsc.store_scatter(ea_v.at[b], [e0 + lane, col_deg], o16)

    plsc.subcore_barrier()

    ebase = wid * EP

    def _issue_idx(g, sb):
        off = ebase + g * C
        pltpu.async_copy(ei_hbm.at[:, pl.ds(off, C)], ei_v.at[sb],
                         idx_sem.at[sb])
        pltpu.async_copy(ea_hbm.at[:, pl.ds(off, C)], eat_v.at[sb],
                         idx_sem.at[sb])

    def _wait_idx(sb):
        pltpu.make_async_copy(ei_hbm.at[:, pl.ds(0, C)], ei_v.at[sb],
                              idx_sem.at[sb]).wait()
        pltpu.make_async_copy(ea_hbm.at[:, pl.ds(0, C)], eat_v.at[sb],
                              idx_sem.at[sb]).wait()

    def _issue_gather(rb, sb):
        pltpu.async_copy(h_hbm.at[ei_v.at[sb, 0]], rows_v.at[rb],
                         gat_sem.at[rb])

    def _wait_gather(rb, sb):
        pltpu.make_async_copy(h_hbm.at[ei_v.at[sb, 0]], rows_v.at[rb],
                              gat_sem.at[rb]).wait()

    def _issue_scatter(rb, sb):
        pltpu.async_copy(rows_v.at[rb], aggh_sh.at[ei_v.at[sb, 1]],
                         sca_sem.at[sb], add=True)
        pltpu.async_copy(ea_v.at[sb], agge_sh.at[ei_v.at[sb, 1]],
                         sca_sem.at[sb], add=True)

    def _drain_scatter(rb, sb):
        pltpu.make_async_copy(rows_v.at[rb], aggh_sh.at[ei_v.at[sb, 1]],
                              sca_sem.at[sb]).wait()
        pltpu.make_async_copy(ea_v.at[sb], agge_sh.at[ei_v.at[sb, 1]],
                              sca_sem.at[sb]).wait()

    def _transpose_ea(sb):
        for e0 in range(0, C, L):
            row_idx = e0 + lane
            for k in range(DE):
                v = eat_v[sb, k, pl.ds(e0, L)]
                plsc.store_scatter(ea_v.at[sb],
                                   [row_idx, jnp.full((L,), k, jnp.int32)],
                                   v)

    _issue_idx(0, 0)
    _issue_idx(1, 1)
    _wait_idx(0)
    _issue_gather(0, 0)

    def _chunk(g, _):
        rb = g % 2
        nrb = (g + 1) % 2
        sb = g % NSLOT

        @pl.when(g > 0)
        def _():
            _drain_scatter(nrb, (g - 1) % NSLOT)

        @pl.when(g + 1 < NCHUNK)
        def _():
            _wait_idx((g + 1) % NSLOT)
            _issue_gather(nrb, (g + 1) % NSLOT)

        _transpose_ea(sb)
        _wait_gather(rb, sb)
        _issue_scatter(rb, sb)

        @pl.when(g + 2 < NCHUNK)
        def _():
            _issue_idx(g + 2, (g + 2) % NSLOT)
        return 0
    lax.fori_loop(0, NCHUNK, _chunk, 0)

    _drain_scatter((NCHUNK - 1) % 2, (NCHUNK - 1) % NSLOT)
    plsc.subcore_barrier()

    rows = pl.ds(s * RPT, RPT)
    pltpu.sync_copy(aggh_sh.at[rows], aggh_hbm.at[c, rows])
    pltpu.sync_copy(agge_sh.at[rows], agge_hbm.at[c, rows])


def _sc_scatter(h, eiT, eaT):
    mesh = plsc.VectorSubcoreMesh(core_axis_name="c", subcore_axis_name="s")
    fn = functools.partial(
        pl.kernel, mesh=mesh,
        out_type=[
            jax.ShapeDtypeStruct((NC, NPAD, D), jnp.float32),
            jax.ShapeDtypeStruct((NC, NPAD, EW), jnp.float32),
        ],
        scratch_types=[
            pltpu.VMEM_SHARED((NPAD, D), jnp.float32),
            pltpu.VMEM_SHARED((NPAD, EW), jnp.float32),
            pltpu.VMEM((NSLOT, 2, C), jnp.int32),
            pltpu.VMEM((NSLOT, DE, C), jnp.float32),
            pltpu.VMEM((NSLOT, C, EW), jnp.float32),
            pltpu.VMEM((2, C, D), jnp.float32),
            pltpu.SemaphoreType.DMA((NSLOT,)),
            pltpu.SemaphoreType.DMA((2,)),
            pltpu.SemaphoreType.DMA((NSLOT,)),
        ],
        compiler_params=pltpu.CompilerParams(use_tc_tiling_on_sc=False,
                                             needs_layout_passes=False),
    )(_sc_body)
    return fn(h, eiT, eaT)


def _mlp_body(h_ref, ah_ref, ae_ref, wea_ref,
              w1_ref, b1_ref, w2_ref, b2_ref, o_ref):
    x = h_ref[...] + ah_ref[0] + ah_ref[1]
    e = ae_ref[0] + ae_ref[1]
    u = x + jnp.dot(e, wea_ref[...], preferred_element_type=jnp.float32)
    v = jnp.dot(u, w1_ref[...], preferred_element_type=jnp.float32) + b1_ref[...]
    v = 0.5 * v * (1.0 + lax.erf(v * 0.7071067811865476))
    o_ref[...] = jnp.dot(v, w2_ref[...], preferred_element_type=jnp.float32) + b2_ref[...]


def _tc_mlp(h, aggh, agge, We_aug, W1, b1, W2, b2):
    R = 1000
    grid = (N // R,)
    return pl.pallas_call(
        _mlp_body,
        grid=grid,
        in_specs=[
            pl.BlockSpec((R, D), lambda i: (i, 0)),
            pl.BlockSpec((NC, R, D), lambda i: (0, i, 0)),
            pl.BlockSpec((NC, R, EW), lambda i: (0, i, 0)),
            pl.BlockSpec((EW, D), lambda i: (0, 0)),
            pl.BlockSpec((D, D), lambda i: (0, 0)),
            pl.BlockSpec((1, D), lambda i: (0, 0)),
            pl.BlockSpec((D, D), lambda i: (0, 0)),
            pl.BlockSpec((1, D), lambda i: (0, 0)),
        ],
        out_specs=pl.BlockSpec((R, D), lambda i: (i, 0)),
        out_shape=jax.ShapeDtypeStruct((N, D), jnp.float32),
    )(h, aggh, agge, We_aug, W1, b1, W2, b2)


def kernel(h, edge_index, edge_attr, We, be, W1, b1, W2, b2):
    eiT = edge_index.T
    eaT = edge_attr.T
    aggh, agge = _sc_scatter(h, eiT, eaT)
    We_aug = jnp.concatenate(
        [We, be[None, :], jnp.zeros((EW - DE - 1, D), jnp.float32)], axis=0)
    return _tc_mlp(h, aggh, agge, We_aug, W1, b1.reshape(1, D),
                   W2, b2.reshape(1, D))
